# Initial kernel scaffold; baseline (speedup 1.0000x reference)
import jax
import jax.numpy as jnp
from jax.experimental import pallas as pl


def kernel(x, edge_index, edge_weight, W_x2h, b_x2h, W_h2h, b_h2h, ln_gamma, ln_beta, num_steps):
    raise NotImplementedError("write your pallas kernel here")


# speedup vs baseline: 1.0547x; 1.0547x over previous
import functools

import jax
import jax.numpy as jnp
from jax.experimental import pallas as pl
from jax.experimental.pallas import tpu as pltpu

N = 10000
D = 128
ROW_BLOCK = 1000


def _gru_ln_body(ns_ref, agg0_ref, agg1_ref, wx_ref, bx_ref, wh_ref, bh_ref,
                 g_ref, b_ref, out_ref):
    res = jnp.maximum(agg0_ref[...] + agg1_ref[...], 0.0)
    gx = jax.lax.dot_general(
        res, wx_ref[...], (((1,), (1,)), ((), ())),
        preferred_element_type=jnp.float32) + bx_ref[...]

    def step(_, hx):
        gh = jax.lax.dot_general(
            hx, wh_ref[...], (((1,), (1,)), ((), ())),
            preferred_element_type=jnp.float32) + bh_ref[...]
        i_r, i_i, i_n = gx[:, :D], gx[:, D:2 * D], gx[:, 2 * D:]
        h_r, h_i, h_n = gh[:, :D], gh[:, D:2 * D], gh[:, 2 * D:]
        rg = jax.nn.sigmoid(i_r + h_r)
        ig = jax.nn.sigmoid(i_i + h_i)
        ng = jnp.tanh(i_n + rg * h_n)
        return ng + ig * (hx - ng)

    hx = jax.lax.fori_loop(0, ns_ref[0], step, jnp.zeros_like(res))
    mean = jnp.mean(hx, axis=-1, keepdims=True)
    var = jnp.mean((hx - mean) ** 2, axis=-1, keepdims=True)
    out_ref[...] = (hx - mean) * jax.lax.rsqrt(var + 1e-5) * g_ref[...] + b_ref[...]


@jax.jit
def _gru_ln(agg0, agg1, W_x2h, b_x2h, W_h2h, b_h2h, ln_gamma, ln_beta, ns):
    n = agg0.shape[0]
    grid = (n // ROW_BLOCK,)
    full = lambda i: (0, 0)
    return pl.pallas_call(
        _gru_ln_body,
        grid=grid,
        in_specs=[
            pl.BlockSpec(memory_space=pltpu.SMEM),
            pl.BlockSpec((ROW_BLOCK, D), lambda i: (i, 0)),
            pl.BlockSpec((ROW_BLOCK, D), lambda i: (i, 0)),
            pl.BlockSpec((3 * D, D), full),
            pl.BlockSpec((1, 3 * D), full),
            pl.BlockSpec((3 * D, D), full),
            pl.BlockSpec((1, 3 * D), full),
            pl.BlockSpec((1, D), full),
            pl.BlockSpec((1, D), full),
        ],
        out_specs=pl.BlockSpec((ROW_BLOCK, D), lambda i: (i, 0)),
        out_shape=jax.ShapeDtypeStruct((n, D), jnp.float32),
    )(ns, agg0, agg1, W_x2h, b_x2h.reshape(1, -1), W_h2h,
      b_h2h.reshape(1, -1), ln_gamma.reshape(1, -1), ln_beta.reshape(1, -1))


def kernel(x, edge_index, edge_weight, W_x2h, b_x2h, W_h2h, b_h2h,
           ln_gamma, ln_beta, num_steps):
    n = x.shape[0]
    src = edge_index[0]
    dst = edge_index[1]
    msg = x[src] * edge_weight[:, None]
    agg = jax.ops.segment_sum(msg, dst, num_segments=n)
    ns = jnp.asarray(num_steps, jnp.int32).reshape(1)
    zero = jnp.zeros_like(agg)
    return _gru_ln(agg, zero, W_x2h, b_x2h, W_h2h, b_h2h, ln_gamma, ln_beta, ns)


# speedup vs baseline: 3.0901x; 2.9300x over previous
import functools

import jax
import jax.numpy as jnp
from jax import lax
from jax.experimental import pallas as pl
from jax.experimental.pallas import tpu as pltpu
from jax.experimental.pallas import tpu_sc as plsc

N = 10000
N_PAD = 10240
D = 128
ROW_BLOCK = 1024

NW = 32
CHUNKS = 80
CW = 128
EPW = CHUNKS * CW
EPAD = NW * EPW
STRIPE = N_PAD // 16


def _spmm_body(x_hbm, src_hbm, dst_hbm, w_hbm, out_hbm,
               src_v, dst_v, w_v, rows_v, acc_s, sem):
    c = lax.axis_index("c")
    s = lax.axis_index("s")
    wid = s * 2 + c
    pltpu.sync_copy(src_hbm.at[wid], src_v)
    pltpu.sync_copy(dst_hbm.at[wid], dst_v)
    pltpu.sync_copy(w_hbm.at[wid], w_v)

    def zrow(e, _):
        for k in range(D // 16):
            rows_v[e, pl.ds(k * 16, 16)] = jnp.zeros((16,), jnp.float32)
        return 0
    lax.fori_loop(0, CW, zrow, 0)
    for t in range(STRIPE // CW):
        pltpu.sync_copy(rows_v, acc_s.at[pl.ds(s * STRIPE + t * CW, CW)])
    plsc.subcore_barrier()

    def chunk(j, _):
        pltpu.async_copy(x_hbm.at[src_v.at[j]], rows_v, sem).wait()

        def grp(g, _):
            wvec = w_v[j, pl.ds(g * 16, 16)]
            for l in range(16):
                w = wvec[l]
                e = g * 16 + l
                for k in range(D // 16):
                    sl = pl.ds(k * 16, 16)
                    rows_v[e, sl] = rows_v[e, sl] * w
            return 0
        lax.fori_loop(0, CW // 16, grp, 0)

        pltpu.sync_copy(rows_v, acc_s.at[dst_v.at[j]], add=True)
        return 0
    lax.fori_loop(0, CHUNKS, chunk, 0)
    plsc.subcore_barrier()

    pltpu.sync_copy(acc_s.at[pl.ds(s * STRIPE, STRIPE)],
                    out_hbm.at[c, pl.ds(s * STRIPE, STRIPE)])


@jax.jit
def _spmm(x, srcp, dstp, wp):
    mesh = plsc.VectorSubcoreMesh(core_axis_name="c", subcore_axis_name="s")
    return pl.kernel(
        _spmm_body,
        out_type=jax.ShapeDtypeStruct((2, N_PAD, D), jnp.float32),
        mesh=mesh,
        scratch_types=[
            pltpu.VMEM((CHUNKS, CW), jnp.int32),
            pltpu.VMEM((CHUNKS, CW), jnp.int32),
            pltpu.VMEM((CHUNKS, CW), jnp.float32),
            pltpu.VMEM((CW, D), jnp.float32),
            pltpu.VMEM_SHARED((N_PAD, D), jnp.float32),
            pltpu.SemaphoreType.DMA,
        ],
    )(x, srcp, dstp, wp)


def _gru_ln_body(ns_ref, agg0_ref, agg1_ref, wx_ref, bx_ref, wh_ref, bh_ref,
                 g_ref, b_ref, out_ref):
    res = jnp.maximum(agg0_ref[...] + agg1_ref[...], 0.0)
    gx = jax.lax.dot_general(
        res, wx_ref[...], (((1,), (1,)), ((), ())),
        preferred_element_type=jnp.float32) + bx_ref[...]

    def step(_, hx):
        gh = jax.lax.dot_general(
            hx, wh_ref[...], (((1,), (1,)), ((), ())),
            preferred_element_type=jnp.float32) + bh_ref[...]
        i_r, i_i, i_n = gx[:, :D], gx[:, D:2 * D], gx[:, 2 * D:]
        h_r, h_i, h_n = gh[:, :D], gh[:, D:2 * D], gh[:, 2 * D:]
        rg = jax.nn.sigmoid(i_r + h_r)
        ig = jax.nn.sigmoid(i_i + h_i)
        ng = jnp.tanh(i_n + rg * h_n)
        return ng + ig * (hx - ng)

    hx = jax.lax.fori_loop(0, ns_ref[0], step, jnp.zeros_like(res))
    mean = jnp.mean(hx, axis=-1, keepdims=True)
    var = jnp.mean((hx - mean) ** 2, axis=-1, keepdims=True)
    out_ref[...] = (hx - mean) * jax.lax.rsqrt(var + 1e-5) * g_ref[...] + b_ref[...]


@jax.jit
def _gru_ln(agg0, agg1, W_x2h, b_x2h, W_h2h, b_h2h, ln_gamma, ln_beta, ns):
    n = agg0.shape[0]
    grid = (n // ROW_BLOCK,)
    full = lambda i: (0, 0)
    return pl.pallas_call(
        _gru_ln_body,
        grid=grid,
        in_specs=[
            pl.BlockSpec(memory_space=pltpu.SMEM),
            pl.BlockSpec((ROW_BLOCK, D), lambda i: (i, 0)),
            pl.BlockSpec((ROW_BLOCK, D), lambda i: (i, 0)),
            pl.BlockSpec((3 * D, D), full),
            pl.BlockSpec((1, 3 * D), full),
            pl.BlockSpec((3 * D, D), full),
            pl.BlockSpec((1, 3 * D), full),
            pl.BlockSpec((1, D), full),
            pl.BlockSpec((1, D), full),
        ],
        out_specs=pl.BlockSpec((ROW_BLOCK, D), lambda i: (i, 0)),
        out_shape=jax.ShapeDtypeStruct((n, D), jnp.float32),
    )(ns, agg0, agg1, W_x2h, b_x2h.reshape(1, -1), W_h2h,
      b_h2h.reshape(1, -1), ln_gamma.reshape(1, -1), ln_beta.reshape(1, -1))


def kernel(x, edge_index, edge_weight, W_x2h, b_x2h, W_h2h, b_h2h,
           ln_gamma, ln_beta, num_steps):
    e = edge_index.shape[1]
    pad = EPAD - e
    srcp = jnp.pad(edge_index[0], (0, pad)).reshape(NW, CHUNKS, CW)
    dstp = jnp.pad(edge_index[1], (0, pad)).reshape(NW, CHUNKS, CW)
    wp = jnp.pad(edge_weight, (0, pad)).reshape(NW, CHUNKS, CW)
    parts = _spmm(x, srcp, dstp, wp)
    ns = jnp.asarray(num_steps, jnp.int32).reshape(1)
    out = _gru_ln(parts[0], parts[1], W_x2h, b_x2h, W_h2h, b_h2h,
                  ln_gamma, ln_beta, ns)
    return out[: x.shape[0]]


# speedup vs baseline: 4.2756x; 1.3836x over previous
import jax
import jax.numpy as jnp
from jax import lax
from jax.experimental import pallas as pl
from jax.experimental.pallas import tpu as pltpu
from jax.experimental.pallas import tpu_sc as plsc

N = 10000
N_PAD = 10240
D = 128
HD = D // 2
ROW_BLOCK = 1024

NT = 16
CHUNKS = 160
CW = 128
EPT = CHUNKS * CW
EPAD = NT * EPT
STRIPE = N_PAD // NT
NBUF = 2


def _spmm_body(xh_hbm, idx_hbm, w_hbm, out_hbm, idx_v, w_v, rows0, rows1,
               acc_s, gsems, ssems):
    rows = (rows0, rows1)
    c = lax.axis_index("c")
    s = lax.axis_index("s")
    pltpu.sync_copy(idx_hbm.at[s], idx_v)
    pltpu.sync_copy(w_hbm.at[s], w_v)

    base = c * N

    def offs(r, _):
        for k in range(CW // 16):
            sl = pl.ds(k * 16, 16)
            idx_v[r, sl] = idx_v[r, sl] + base
        return 0
    lax.fori_loop(0, CHUNKS, offs, 0)

    def zrow(e, _):
        for k in range(HD // 16):
            rows0[e, pl.ds(k * 16, 16)] = jnp.zeros((16,), jnp.float32)
        return 0
    lax.fori_loop(0, CW, zrow, 0)

    def zacc(t, _):
        pltpu.sync_copy(rows0, acc_s.at[pl.ds(s * STRIPE + t * CW, CW)])
        return 0
    lax.fori_loop(0, STRIPE // CW, zacc, 0)
    plsc.subcore_barrier()

    def scale(b, j):
        def grp(g, _):
            wvec = w_v[j, pl.ds(g * 16, 16)]
            for l in range(16):
                w = wvec[l]
                e = g * 16 + l
                for k in range(HD // 16):
                    sl = pl.ds(k * 16, 16)
                    rows[b][e, sl] = rows[b][e, sl] * w
            return 0
        lax.fori_loop(0, CW // 16, grp, 0)

    for b in range(NBUF):
        pltpu.async_copy(xh_hbm.at[idx_v.at[b]], rows[b], gsems.at[b])

    T = CHUNKS // NBUF

    def super_chunk(t, _):
        scats = []
        for b in range(NBUF):
            j = t * NBUF + b
            pltpu.make_async_copy(xh_hbm.at[idx_v.at[j]], rows[b],
                                  gsems.at[b]).wait()
            scale(b, j)
            scats.append(pltpu.async_copy(
                rows[b], acc_s.at[idx_v.at[CHUNKS + j]], ssems.at[b],
                add=True))
        for b in range(NBUF):
            scats[b].wait()

            @pl.when(t < T - 1)
            def _():
                jn = (t + 1) * NBUF + b
                pltpu.async_copy(xh_hbm.at[idx_v.at[jn]], rows[b],
                                 gsems.at[b])
        return 0
    lax.fori_loop(0, T, super_chunk, 0)
    plsc.subcore_barrier()

    pltpu.sync_copy(acc_s.at[pl.ds(s * STRIPE, STRIPE)],
                    out_hbm.at[c, pl.ds(s * STRIPE, STRIPE)])


@jax.jit
def _spmm(xh, idxpack, wp):
    mesh = plsc.VectorSubcoreMesh(core_axis_name="c", subcore_axis_name="s")
    return pl.kernel(
        _spmm_body,
        out_type=jax.ShapeDtypeStruct((2, N_PAD, HD), jnp.float32),
        mesh=mesh,
        compiler_params=pltpu.CompilerParams(use_tc_tiling_on_sc=False),
        scratch_types=[
            pltpu.VMEM((2 * CHUNKS, CW), jnp.int32),
            pltpu.VMEM((CHUNKS, CW), jnp.float32),
            pltpu.VMEM((CW, HD), jnp.float32),
            pltpu.VMEM((CW, HD), jnp.float32),
            pltpu.VMEM_SHARED((N_PAD, HD), jnp.float32),
            pltpu.SemaphoreType.DMA((NBUF,)),
            pltpu.SemaphoreType.DMA((NBUF,)),
        ],
    )(xh, idxpack, wp)


def _gru_ln_body(ns_ref, agg0_ref, agg1_ref, wx_ref, bx_ref, wh_ref, bh_ref,
                 g_ref, b_ref, out_ref):
    agg = jnp.concatenate([agg0_ref[...], agg1_ref[...]], axis=1)
    res = jnp.maximum(agg, 0.0)
    gx = jax.lax.dot_general(
        res, wx_ref[...], (((1,), (1,)), ((), ())),
        preferred_element_type=jnp.float32) + bx_ref[...]

    def step(_, hx):
        gh = jax.lax.dot_general(
            hx, wh_ref[...], (((1,), (1,)), ((), ())),
            preferred_element_type=jnp.float32) + bh_ref[...]
        i_r, i_i, i_n = gx[:, :D], gx[:, D:2 * D], gx[:, 2 * D:]
        h_r, h_i, h_n = gh[:, :D], gh[:, D:2 * D], gh[:, 2 * D:]
        rg = jax.nn.sigmoid(i_r + h_r)
        ig = jax.nn.sigmoid(i_i + h_i)
        ng = jnp.tanh(i_n + rg * h_n)
        return ng + ig * (hx - ng)

    hx = jax.lax.fori_loop(0, ns_ref[0], step, jnp.zeros_like(res))
    mean = jnp.mean(hx, axis=-1, keepdims=True)
    var = jnp.mean((hx - mean) ** 2, axis=-1, keepdims=True)
    out_ref[...] = (hx - mean) * jax.lax.rsqrt(var + 1e-5) * g_ref[...] + b_ref[...]


@jax.jit
def _gru_ln(agg0, agg1, W_x2h, b_x2h, W_h2h, b_h2h, ln_gamma, ln_beta, ns):
    n = agg0.shape[0]
    grid = (n // ROW_BLOCK,)
    full = lambda i: (0, 0)
    return pl.pallas_call(
        _gru_ln_body,
        grid=grid,
        in_specs=[
            pl.BlockSpec(memory_space=pltpu.SMEM),
            pl.BlockSpec((ROW_BLOCK, HD), lambda i: (i, 0)),
            pl.BlockSpec((ROW_BLOCK, HD), lambda i: (i, 0)),
            pl.BlockSpec((3 * D, D), full),
            pl.BlockSpec((1, 3 * D), full),
            pl.BlockSpec((3 * D, D), full),
            pl.BlockSpec((1, 3 * D), full),
            pl.BlockSpec((1, D), full),
            pl.BlockSpec((1, D), full),
        ],
        out_specs=pl.BlockSpec((ROW_BLOCK, D), lambda i: (i, 0)),
        out_shape=jax.ShapeDtypeStruct((n, D), jnp.float32),
    )(ns, agg0, agg1, W_x2h, b_x2h.reshape(1, -1), W_h2h,
      b_h2h.reshape(1, -1), ln_gamma.reshape(1, -1), ln_beta.reshape(1, -1))


def kernel(x, edge_index, edge_weight, W_x2h, b_x2h, W_h2h, b_h2h,
           ln_gamma, ln_beta, num_steps):
    e = edge_index.shape[1]
    pad = EPAD - e
    srcp = jnp.pad(edge_index[0], (0, pad)).reshape(NT, CHUNKS, CW)
    dstp = jnp.pad(edge_index[1], (0, pad)).reshape(NT, CHUNKS, CW)
    wp = jnp.pad(edge_weight, (0, pad)).reshape(NT, CHUNKS, CW)
    idxpack = jnp.concatenate([srcp, dstp], axis=1)
    xh = jnp.concatenate([x[:, :HD], x[:, HD:]], axis=0)
    parts = _spmm(xh, idxpack, wp)
    ns = jnp.asarray(num_steps, jnp.int32).reshape(1)
    out = _gru_ln(parts[0], parts[1], W_x2h, b_x2h, W_h2h, b_h2h,
                  ln_gamma, ln_beta, ns)
    return out[: x.shape[0]]
